# baseline (speedup 1.0000x reference)
import functools

import jax
import jax.numpy as jnp
from jax.experimental import pallas as pl
from jax.experimental.pallas import tpu as pltpu

_NUM_CLASSES = 81
_EPS = 1e-3


def _bbox_head_kernel(
    a_ref, w1_ref, w2_ref, wc_ref, wd_ref,
    b1_ref, g1_ref, be1_ref, b2_ref, g2_ref, be2_ref, bc_ref, bd_ref,
    logits_ref, probs_ref, deltas_ref,
    acc_ref, *, num_k_blocks,
):
    k = pl.program_id(0)
    a = a_ref[...].astype(jnp.bfloat16)
    w = w1_ref[...].astype(jnp.bfloat16)
    part = jnp.dot(a, w, preferred_element_type=jnp.float32)

    @pl.when(k == 0)
    def _init():
        acc_ref[...] = part

    @pl.when(k > 0)
    def _accum():
        acc_ref[...] += part

    @pl.when(k == num_k_blocks - 1)
    def _epilogue():
        x1 = acc_ref[...] + b1_ref[...]
        mean1 = jnp.mean(x1, axis=0, keepdims=True)
        var1 = jnp.mean((x1 - mean1) ** 2, axis=0, keepdims=True)
        h1 = g1_ref[...] * (x1 - mean1) / jnp.sqrt(var1 + _EPS) + be1_ref[...]
        h1 = jnp.maximum(h1, 0.0).astype(jnp.bfloat16)

        w2 = w2_ref[...].astype(jnp.bfloat16)
        x2 = jnp.dot(h1, w2, preferred_element_type=jnp.float32) + b2_ref[...]
        mean2 = jnp.mean(x2, axis=0, keepdims=True)
        var2 = jnp.mean((x2 - mean2) ** 2, axis=0, keepdims=True)
        h2 = g2_ref[...] * (x2 - mean2) / jnp.sqrt(var2 + _EPS) + be2_ref[...]
        h2 = jnp.maximum(h2, 0.0).astype(jnp.bfloat16)

        wc = wc_ref[...].astype(jnp.bfloat16)
        logits = jnp.dot(h2, wc, preferred_element_type=jnp.float32) + bc_ref[...]
        logits_ref[...] = logits
        m = jnp.max(logits, axis=1, keepdims=True)
        e = jnp.exp(logits - m)
        probs_ref[...] = e / jnp.sum(e, axis=1, keepdims=True)

        wd = wd_ref[...].astype(jnp.bfloat16)
        deltas_ref[...] = (
            jnp.dot(h2, wd, preferred_element_type=jnp.float32) + bd_ref[...]
        )


def kernel(pooled_rois, W1, b1, gamma1, beta1, W2, b2, gamma2, beta2, Wc, bc, Wd, bd):
    n = pooled_rois.shape[0]
    a = pooled_rois.reshape(n, -1)
    w1 = W1.reshape(-1, W1.shape[-1])
    k_total = a.shape[1]
    kb = 896
    num_k_blocks = k_total // kb
    nc = Wc.shape[1]
    nd = Wd.shape[1]

    row = lambda v: v.reshape(1, -1)
    full = lambda arr: pl.BlockSpec(arr.shape, lambda k: (0,) * arr.ndim)

    logits, probs, deltas = pl.pallas_call(
        functools.partial(_bbox_head_kernel, num_k_blocks=num_k_blocks),
        grid=(num_k_blocks,),
        in_specs=[
            pl.BlockSpec((n, kb), lambda k: (0, k)),
            pl.BlockSpec((kb, w1.shape[1]), lambda k: (k, 0)),
            full(W2), full(Wc), full(Wd),
            full(row(b1)), full(row(gamma1)), full(row(beta1)),
            full(row(b2)), full(row(gamma2)), full(row(beta2)),
            full(row(bc)), full(row(bd)),
        ],
        out_specs=[
            pl.BlockSpec((n, nc), lambda k: (0, 0)),
            pl.BlockSpec((n, nc), lambda k: (0, 0)),
            pl.BlockSpec((n, nd), lambda k: (0, 0)),
        ],
        out_shape=[
            jax.ShapeDtypeStruct((n, nc), jnp.float32),
            jax.ShapeDtypeStruct((n, nc), jnp.float32),
            jax.ShapeDtypeStruct((n, nd), jnp.float32),
        ],
        scratch_shapes=[pltpu.VMEM((n, w1.shape[1]), jnp.float32)],
        compiler_params=pltpu.CompilerParams(
            dimension_semantics=("arbitrary",),
        ),
    )(
        a, w1, W2, Wc, Wd,
        row(b1), row(gamma1), row(beta1),
        row(b2), row(gamma2), row(beta2),
        row(bc), row(bd),
    )
    return (logits, probs, deltas.reshape(n, _NUM_CLASSES, 4))


# speedup vs baseline: 1.2274x; 1.2274x over previous
import functools

import jax
import jax.numpy as jnp
from jax.experimental import pallas as pl
from jax.experimental.pallas import tpu as pltpu

_NUM_CLASSES = 81
_EPS = 1e-3


def _bbox_head_kernel(
    a_ref, w1_ref, w2_ref, wc_ref, wd_ref,
    b1_ref, g1_ref, be1_ref, b2_ref, g2_ref, be2_ref, bc_ref, bd_ref,
    logits_ref, probs_ref, deltas_ref,
    acc_ref, *, num_i, num_nb, nb,
):
    i = pl.program_id(0)
    n = pl.program_id(1)
    a = a_ref[...].reshape(nb, 7 * 256).astype(jnp.bfloat16)
    w = w1_ref[...].astype(jnp.bfloat16)
    part = jnp.dot(a, w, preferred_element_type=jnp.float32)
    rows = pl.ds(pl.multiple_of(n * nb, 8), nb)

    @pl.when(i == 0)
    def _init():
        acc_ref[rows, :] = part

    @pl.when(i > 0)
    def _accum():
        acc_ref[rows, :] += part

    @pl.when((i == num_i - 1) & (n == num_nb - 1))
    def _epilogue():
        x1 = acc_ref[...] + b1_ref[...]
        mean1 = jnp.mean(x1, axis=0, keepdims=True)
        var1 = jnp.mean((x1 - mean1) ** 2, axis=0, keepdims=True)
        h1 = g1_ref[...] * (x1 - mean1) / jnp.sqrt(var1 + _EPS) + be1_ref[...]
        h1 = jnp.maximum(h1, 0.0).astype(jnp.bfloat16)

        w2 = w2_ref[...].astype(jnp.bfloat16)
        x2 = jnp.dot(h1, w2, preferred_element_type=jnp.float32) + b2_ref[...]
        mean2 = jnp.mean(x2, axis=0, keepdims=True)
        var2 = jnp.mean((x2 - mean2) ** 2, axis=0, keepdims=True)
        h2 = g2_ref[...] * (x2 - mean2) / jnp.sqrt(var2 + _EPS) + be2_ref[...]
        h2 = jnp.maximum(h2, 0.0).astype(jnp.bfloat16)

        wc = wc_ref[...].astype(jnp.bfloat16)
        logits = jnp.dot(h2, wc, preferred_element_type=jnp.float32) + bc_ref[...]
        logits_ref[...] = logits
        m = jnp.max(logits, axis=1, keepdims=True)
        e = jnp.exp(logits - m)
        probs_ref[...] = e / jnp.sum(e, axis=1, keepdims=True)

        wd = wd_ref[...].astype(jnp.bfloat16)
        deltas_ref[...] = (
            jnp.dot(h2, wd, preferred_element_type=jnp.float32) + bd_ref[...]
        )


def kernel(pooled_rois, W1, b1, gamma1, beta1, W2, b2, gamma2, beta2, Wc, bc, Wd, bd):
    n = pooled_rois.shape[0]
    w1 = W1.reshape(-1, W1.shape[-1])
    num_i = 7
    kb = 7 * 256
    num_nb = 5
    nb = n // num_nb
    nc = Wc.shape[1]
    nd = Wd.shape[1]

    row = lambda v: v.reshape(1, -1)
    full = lambda arr: pl.BlockSpec(arr.shape, lambda i, j: (0,) * arr.ndim)

    logits, probs, deltas = pl.pallas_call(
        functools.partial(_bbox_head_kernel, num_i=num_i, num_nb=num_nb, nb=nb),
        grid=(num_i, num_nb),
        in_specs=[
            pl.BlockSpec((nb, 1, 7, 256), lambda i, j: (j, i, 0, 0)),
            pl.BlockSpec((kb, w1.shape[1]), lambda i, j: (i, 0)),
            full(W2), full(Wc), full(Wd),
            full(row(b1)), full(row(gamma1)), full(row(beta1)),
            full(row(b2)), full(row(gamma2)), full(row(beta2)),
            full(row(bc)), full(row(bd)),
        ],
        out_specs=[
            pl.BlockSpec((n, nc), lambda i, j: (0, 0)),
            pl.BlockSpec((n, nc), lambda i, j: (0, 0)),
            pl.BlockSpec((n, nd), lambda i, j: (0, 0)),
        ],
        out_shape=[
            jax.ShapeDtypeStruct((n, nc), jnp.float32),
            jax.ShapeDtypeStruct((n, nc), jnp.float32),
            jax.ShapeDtypeStruct((n, nd), jnp.float32),
        ],
        scratch_shapes=[pltpu.VMEM((n, w1.shape[1]), jnp.float32)],
        compiler_params=pltpu.CompilerParams(
            dimension_semantics=("arbitrary", "arbitrary"),
        ),
    )(
        pooled_rois, w1, W2, Wc, Wd,
        row(b1), row(gamma1), row(beta1),
        row(b2), row(gamma2), row(beta2),
        row(bc), row(bd),
    )
    return (logits, probs, deltas.reshape(n, _NUM_CLASSES, 4))


# speedup vs baseline: 1.9723x; 1.6069x over previous
import jax
import jax.numpy as jnp
from jax.experimental import pallas as pl
from jax.experimental.pallas import tpu as pltpu

_NUM_CLASSES = 81
_EPS = 1e-3


def _bbox_head_kernel(
    a_ref, w1_ref, w2_ref, wc_ref, wd_ref,
    b1_ref, g1_ref, be1_ref, b2_ref, g2_ref, be2_ref, bc_ref, bd_ref,
    logits_ref, probs_ref, deltas_ref,
    acc_ref,
):
    i = pl.program_id(0)
    j = pl.program_id(1)
    a = a_ref[0, 0].astype(jnp.bfloat16)
    w = w1_ref[0, 0].astype(jnp.bfloat16)
    part = jnp.dot(a, w, preferred_element_type=jnp.float32)

    @pl.when((i == 0) & (j == 0))
    def _init():
        acc_ref[...] = part

    @pl.when((i > 0) | (j > 0))
    def _accum():
        acc_ref[...] += part

    @pl.when((i == 6) & (j == 6))
    def _epilogue():
        x1 = acc_ref[...] + b1_ref[...]
        mean1 = jnp.mean(x1, axis=0, keepdims=True)
        var1 = jnp.mean((x1 - mean1) ** 2, axis=0, keepdims=True)
        h1 = g1_ref[...] * (x1 - mean1) / jnp.sqrt(var1 + _EPS) + be1_ref[...]
        h1 = jnp.maximum(h1, 0.0).astype(jnp.bfloat16)

        w2 = w2_ref[...].astype(jnp.bfloat16)
        x2 = jnp.dot(h1, w2, preferred_element_type=jnp.float32) + b2_ref[...]
        mean2 = jnp.mean(x2, axis=0, keepdims=True)
        var2 = jnp.mean((x2 - mean2) ** 2, axis=0, keepdims=True)
        h2 = g2_ref[...] * (x2 - mean2) / jnp.sqrt(var2 + _EPS) + be2_ref[...]
        h2 = jnp.maximum(h2, 0.0).astype(jnp.bfloat16)

        wc = wc_ref[...].astype(jnp.bfloat16)
        logits = jnp.dot(h2, wc, preferred_element_type=jnp.float32) + bc_ref[...]
        logits_ref[...] = logits
        m = jnp.max(logits, axis=1, keepdims=True)
        e = jnp.exp(logits - m)
        probs_ref[...] = e / jnp.sum(e, axis=1, keepdims=True)

        wd = wd_ref[...].astype(jnp.bfloat16)
        deltas_ref[...] = (
            jnp.dot(h2, wd, preferred_element_type=jnp.float32) + bd_ref[...]
        )


def kernel(pooled_rois, W1, b1, gamma1, beta1, W2, b2, gamma2, beta2, Wc, bc, Wd, bd):
    n = pooled_rois.shape[0]
    a_t = jnp.transpose(pooled_rois, (1, 2, 0, 3))
    nc = Wc.shape[1]
    nd = Wd.shape[1]

    row = lambda v: v.reshape(1, -1)
    full = lambda arr: pl.BlockSpec(arr.shape, lambda i, j: (0,) * arr.ndim)

    logits, probs, deltas = pl.pallas_call(
        _bbox_head_kernel,
        grid=(7, 7),
        in_specs=[
            pl.BlockSpec((1, 1, n, 256), lambda i, j: (i, j, 0, 0)),
            pl.BlockSpec((1, 1, 256, 1024), lambda i, j: (i, j, 0, 0)),
            full(W2), full(Wc), full(Wd),
            full(row(b1)), full(row(gamma1)), full(row(beta1)),
            full(row(b2)), full(row(gamma2)), full(row(beta2)),
            full(row(bc)), full(row(bd)),
        ],
        out_specs=[
            pl.BlockSpec((n, nc), lambda i, j: (0, 0)),
            pl.BlockSpec((n, nc), lambda i, j: (0, 0)),
            pl.BlockSpec((n, nd), lambda i, j: (0, 0)),
        ],
        out_shape=[
            jax.ShapeDtypeStruct((n, nc), jnp.float32),
            jax.ShapeDtypeStruct((n, nc), jnp.float32),
            jax.ShapeDtypeStruct((n, nd), jnp.float32),
        ],
        scratch_shapes=[pltpu.VMEM((n, 1024), jnp.float32)],
        compiler_params=pltpu.CompilerParams(
            dimension_semantics=("arbitrary", "arbitrary"),
        ),
    )(
        a_t, W1, W2, Wc, Wd,
        row(b1), row(gamma1), row(beta1),
        row(b2), row(gamma2), row(beta2),
        row(bc), row(bd),
    )
    return (logits, probs, deltas.reshape(n, _NUM_CLASSES, 4))


# speedup vs baseline: 2.3699x; 1.2016x over previous
import jax
import jax.numpy as jnp
from jax.experimental import pallas as pl
from jax.experimental.pallas import tpu as pltpu

_NUM_CLASSES = 81
_EPS = 1e-3


def _bbox_head_kernel(
    a_ref, w1_ref, w2_ref, wc_ref, wd_ref,
    b1_ref, g1_ref, be1_ref, b2_ref, g2_ref, be2_ref, bc_ref, bd_ref,
    logits_ref, probs_ref, deltas_ref,
    acc_ref,
):
    i = pl.program_id(0)
    n = pl.program_id(1)
    part = None
    for jj in range(7):
        a = a_ref[0, jj].astype(jnp.bfloat16)
        w = w1_ref[0, jj].astype(jnp.bfloat16)
        d = jnp.dot(a, w, preferred_element_type=jnp.float32)
        part = d if part is None else part + d
    nb = a_ref.shape[2]
    rows = pl.ds(pl.multiple_of(n * nb, 8), nb)

    @pl.when(i == 0)
    def _init():
        acc_ref[rows, :] = part

    @pl.when(i > 0)
    def _accum():
        acc_ref[rows, :] += part

    @pl.when((i == 6) & (n == pl.num_programs(1) - 1))
    def _epilogue():
        x1 = acc_ref[...] + b1_ref[...]
        mean1 = jnp.mean(x1, axis=0, keepdims=True)
        var1 = jnp.mean((x1 - mean1) ** 2, axis=0, keepdims=True)
        h1 = g1_ref[...] * (x1 - mean1) / jnp.sqrt(var1 + _EPS) + be1_ref[...]
        h1 = jnp.maximum(h1, 0.0).astype(jnp.bfloat16)

        w2 = w2_ref[...].astype(jnp.bfloat16)
        x2 = jnp.dot(h1, w2, preferred_element_type=jnp.float32) + b2_ref[...]
        mean2 = jnp.mean(x2, axis=0, keepdims=True)
        var2 = jnp.mean((x2 - mean2) ** 2, axis=0, keepdims=True)
        h2 = g2_ref[...] * (x2 - mean2) / jnp.sqrt(var2 + _EPS) + be2_ref[...]
        h2 = jnp.maximum(h2, 0.0).astype(jnp.bfloat16)

        wc = wc_ref[...].astype(jnp.bfloat16)
        logits = jnp.dot(h2, wc, preferred_element_type=jnp.float32) + bc_ref[...]
        logits_ref[...] = logits
        m = jnp.max(logits, axis=1, keepdims=True)
        e = jnp.exp(logits - m)
        probs_ref[...] = e / jnp.sum(e, axis=1, keepdims=True)

        wd = wd_ref[...].astype(jnp.bfloat16)
        deltas_ref[...] = (
            jnp.dot(h2, wd, preferred_element_type=jnp.float32) + bd_ref[...]
        )


def kernel(pooled_rois, W1, b1, gamma1, beta1, W2, b2, gamma2, beta2, Wc, bc, Wd, bd):
    n = pooled_rois.shape[0]
    a_t = jnp.transpose(pooled_rois, (1, 2, 0, 3))
    nc = Wc.shape[1]
    nd = Wd.shape[1]

    row = lambda v: v.reshape(1, -1)
    full = lambda arr: pl.BlockSpec(arr.shape, lambda i, j: (0,) * arr.ndim)

    logits, probs, deltas = pl.pallas_call(
        _bbox_head_kernel,
        grid=(7, 5),
        in_specs=[
            pl.BlockSpec((1, 7, n // 5, 256), lambda i, j: (i, 0, j, 0)),
            pl.BlockSpec((1, 7, 256, 1024), lambda i, j: (i, 0, 0, 0)),
            full(W2), full(Wc), full(Wd),
            full(row(b1)), full(row(gamma1)), full(row(beta1)),
            full(row(b2)), full(row(gamma2)), full(row(beta2)),
            full(row(bc)), full(row(bd)),
        ],
        out_specs=[
            pl.BlockSpec((n, nc), lambda i, j: (0, 0)),
            pl.BlockSpec((n, nc), lambda i, j: (0, 0)),
            pl.BlockSpec((n, nd), lambda i, j: (0, 0)),
        ],
        out_shape=[
            jax.ShapeDtypeStruct((n, nc), jnp.float32),
            jax.ShapeDtypeStruct((n, nc), jnp.float32),
            jax.ShapeDtypeStruct((n, nd), jnp.float32),
        ],
        scratch_shapes=[pltpu.VMEM((n, 1024), jnp.float32)],
        compiler_params=pltpu.CompilerParams(
            dimension_semantics=("arbitrary", "arbitrary"),
        ),
    )(
        a_t, W1, W2, Wc, Wd,
        row(b1), row(gamma1), row(beta1),
        row(b2), row(gamma2), row(beta2),
        row(bc), row(bd),
    )
    return (logits, probs, deltas.reshape(n, _NUM_CLASSES, 4))
